# baseline (speedup 1.0000x reference)
import jax
import jax.numpy as jnp
from jax.experimental import pallas as pl
from jax.experimental.pallas import tpu as pltpu

N = 10000
F_IN = 128
F_OUT = 128
BM = 200
NSTEPS = N // BM
NBUF = 4


def _gcn_kernel(x_ref, w_ref, adj_hbm, out_ref, xw_ref, bufs, sems):
    for b in range(NBUF):
        pltpu.make_async_copy(
            adj_hbm.at[pl.ds(b * BM, BM), :], bufs.at[b], sems.at[b]
        ).start()

    xw_ref[...] = jnp.dot(x_ref[...], w_ref[...],
                          preferred_element_type=jnp.float32)

    def body(i, carry):
        slot = jax.lax.rem(i, NBUF)
        pltpu.make_async_copy(
            adj_hbm.at[pl.ds(i * BM, BM), :], bufs.at[slot], sems.at[slot]
        ).wait()
        acc = jnp.dot(bufs[slot], xw_ref[...],
                      preferred_element_type=jnp.float32)
        out_ref[pl.ds(i * BM, BM), :] = jnp.maximum(acc, 0.0)

        @pl.when(i + NBUF < NSTEPS)
        def _():
            pltpu.make_async_copy(
                adj_hbm.at[pl.ds((i + NBUF) * BM, BM), :],
                bufs.at[slot], sems.at[slot]
            ).start()

        return carry

    jax.lax.fori_loop(0, NSTEPS, body, 0)


def kernel(input, adj, weight):
    return pl.pallas_call(
        _gcn_kernel,
        in_specs=[
            pl.BlockSpec(memory_space=pltpu.VMEM),
            pl.BlockSpec(memory_space=pltpu.VMEM),
            pl.BlockSpec(memory_space=pltpu.MemorySpace.HBM),
        ],
        out_specs=pl.BlockSpec(memory_space=pltpu.VMEM),
        out_shape=jax.ShapeDtypeStruct((N, F_OUT), jnp.float32),
        scratch_shapes=[
            pltpu.VMEM((N, F_OUT), jnp.float32),
            pltpu.VMEM((NBUF, BM, N), jnp.float32),
            pltpu.SemaphoreType.DMA((NBUF,)),
        ],
    )(input, weight, adj)


# speedup vs baseline: 1.0052x; 1.0052x over previous
import jax
import jax.numpy as jnp
from jax.experimental import pallas as pl
from jax.experimental.pallas import tpu as pltpu

N = 10000
F_IN = 128
F_OUT = 128
BM = 400
NSTEPS = N // BM
NBUF = 2


def _gcn_kernel(x_ref, w_ref, adj_hbm, out_ref, xw_ref, bufs, sems):
    for b in range(NBUF):
        pltpu.make_async_copy(
            adj_hbm.at[pl.ds(b * BM, BM), :], bufs.at[b], sems.at[b]
        ).start()

    xw_ref[...] = jnp.dot(x_ref[...], w_ref[...],
                          preferred_element_type=jnp.float32)

    def body(i, carry):
        slot = jax.lax.rem(i, NBUF)
        pltpu.make_async_copy(
            adj_hbm.at[pl.ds(i * BM, BM), :], bufs.at[slot], sems.at[slot]
        ).wait()
        acc = jnp.dot(bufs[slot], xw_ref[...],
                      preferred_element_type=jnp.float32)
        out_ref[pl.ds(i * BM, BM), :] = jnp.maximum(acc, 0.0)

        @pl.when(i + NBUF < NSTEPS)
        def _():
            pltpu.make_async_copy(
                adj_hbm.at[pl.ds((i + NBUF) * BM, BM), :],
                bufs.at[slot], sems.at[slot]
            ).start()

        return carry

    jax.lax.fori_loop(0, NSTEPS, body, 0)


def kernel(input, adj, weight):
    return pl.pallas_call(
        _gcn_kernel,
        in_specs=[
            pl.BlockSpec(memory_space=pltpu.VMEM),
            pl.BlockSpec(memory_space=pltpu.VMEM),
            pl.BlockSpec(memory_space=pltpu.MemorySpace.HBM),
        ],
        out_specs=pl.BlockSpec(memory_space=pltpu.VMEM),
        out_shape=jax.ShapeDtypeStruct((N, F_OUT), jnp.float32),
        scratch_shapes=[
            pltpu.VMEM((N, F_OUT), jnp.float32),
            pltpu.VMEM((NBUF, BM, N), jnp.float32),
            pltpu.SemaphoreType.DMA((NBUF,)),
        ],
    )(input, weight, adj)
